# baseline (speedup 1.0000x reference)
import functools

import jax
import jax.numpy as jnp
from jax import lax
from jax.experimental import pallas as pl
from jax.experimental.pallas import tpu as pltpu
from jax.experimental.pallas import tpu_sc as plsc

_BATCH = 1024
_LAT_DIM = 64
_VOCAB = 256
_CUR_DIM = 32

_NC = 2
_NS = 16
_L = 16
_BBR = 64
_VHW = 128
_NG = _BBR // _L
_NK = _VHW // _L


def _bounds_body(sm_hbm, low_hbm, upp_hbm, sj_hbm, out_low_hbm, out_upp_hbm,
                 p3, sj_v, acc_v, res_v, low_v, upp_v, olow_v, oupp_v,
                 shared, sem):
    c = lax.axis_index("c")
    s = lax.axis_index("s")
    vh = s % 2
    bbw = s // 2 + c * (_NS // 2)
    base = bbw * _BBR
    iota = lax.iota(jnp.int32, _L)
    zero = jnp.zeros((_L,), jnp.float32)

    copy = pltpu.async_copy(
        sm_hbm.at[pl.ds(base, _BBR), pl.ds(_CUR_DIM, 8), pl.ds(vh * _VHW, _VHW)],
        p3, sem)
    pltpu.sync_copy(sj_hbm.at[pl.ds(base, _BBR)], sj_v)
    copy.wait()

    accL = [zero for _ in range(_NG)]
    accA = [zero for _ in range(_NG)]
    sjg = [sj_v[pl.ds(g * _L, _L)] for g in range(_NG)]
    for r in range(_BBR):
        vs = jnp.full((_L,), sjg[r // _L][r % _L], jnp.int32)
        alo = zero
        aat = zero
        for k in range(_NK):
            p = p3[r, 0, pl.ds(k * _L, _L)]
            vidx = iota + (vh * _VHW + k * _L)
            alo = alo + jnp.where(vidx < vs, p, zero)
            aat = aat + jnp.where(vidx == vs, p, zero)
        clo = jnp.full((_L,), jnp.sum(alo), jnp.float32)
        cat = jnp.full((_L,), jnp.sum(aat), jnp.float32)
        g, l = divmod(r, _L)
        lane = iota == l
        accL[g] = accL[g] + jnp.where(lane, clo, zero)
        accA[g] = accA[g] + jnp.where(lane, cat, zero)
    for g in range(_NG):
        acc_v[0, pl.ds(g * _L, _L)] = accL[g]
        acc_v[1, pl.ds(g * _L, _L)] = accA[g]

    pltpu.sync_copy(acc_v, shared.at[s])
    plsc.subcore_barrier()

    @pl.when(vh == 0)
    def _():
        pltpu.sync_copy(shared.at[s + 1], res_v)
        pltpu.sync_copy(low_hbm.at[pl.ds(base, _BBR)], low_v)
        pltpu.sync_copy(upp_hbm.at[pl.ds(base, _BBR)], upp_v)
        for g in range(_NG):
            cdf = accL[g] + res_v[0, pl.ds(g * _L, _L)]
            pat = accA[g] + res_v[1, pl.ds(g * _L, _L)]
            low = low_v[pl.ds(g * _L, _L)]
            upp = upp_v[pl.ds(g * _L, _L)]
            rng = upp - low
            olow_v[pl.ds(g * _L, _L)] = low + rng * cdf
            oupp_v[pl.ds(g * _L, _L)] = low + rng * (cdf + pat)
        pltpu.sync_copy(olow_v, out_low_hbm.at[pl.ds(base, _BBR)])
        pltpu.sync_copy(oupp_v, out_upp_hbm.at[pl.ds(base, _BBR)])


_sc_update_bounds = functools.partial(
    pl.kernel,
    mesh=plsc.VectorSubcoreMesh(core_axis_name="c", subcore_axis_name="s"),
    compiler_params=pltpu.CompilerParams(use_tc_tiling_on_sc=True,
                                         needs_layout_passes=False),
    out_type=(jax.ShapeDtypeStruct((_BATCH,), jnp.float32),
              jax.ShapeDtypeStruct((_BATCH,), jnp.float32)),
    scratch_types=[
        pltpu.VMEM((_BBR, 8, _VHW), jnp.float32),
        pltpu.VMEM((_BBR,), jnp.int32),
        pltpu.VMEM((2, _BBR), jnp.float32),
        pltpu.VMEM((2, _BBR), jnp.float32),
        pltpu.VMEM((_BBR,), jnp.float32),
        pltpu.VMEM((_BBR,), jnp.float32),
        pltpu.VMEM((_BBR,), jnp.float32),
        pltpu.VMEM((_BBR,), jnp.float32),
        pltpu.VMEM_SHARED((_NS, 2, _BBR), jnp.float32),
        pltpu.SemaphoreType.DMA,
    ],
)(_bounds_body)


def kernel(low_bound, upp_bound, softmax, s_j):
    sj = s_j.astype(jnp.int32)
    new_low, new_upp = _sc_update_bounds(softmax, low_bound, upp_bound, sj)
    return (new_low, new_upp)


# speedup vs baseline: 1.0520x; 1.0520x over previous
import functools

import jax
import jax.numpy as jnp
from jax import lax
from jax.experimental import pallas as pl
from jax.experimental.pallas import tpu as pltpu
from jax.experimental.pallas import tpu_sc as plsc

_BATCH = 1024
_LAT_DIM = 64
_VOCAB = 256
_CUR_DIM = 32

_NC = 2
_NS = 16
_L = 16
_NW = _NC * _NS
_BPW = _BATCH // _NW
_BLK = 128
_NG = _BPW // _L


def _slice_t_body(src_ref, dst_ref):
    dst_ref[...] = src_ref[:, _CUR_DIM % 8, :].T


_extract_t = pl.pallas_call(
    _slice_t_body,
    grid=(1,),
    in_specs=[pl.BlockSpec((_BATCH, 8, _VOCAB), lambda i: (0, _CUR_DIM // 8, 0))],
    out_specs=pl.BlockSpec((_VOCAB, _BATCH), lambda i: (0, 0)),
    out_shape=jax.ShapeDtypeStruct((_VOCAB, _BATCH), jnp.float32),
)


def _bounds_body(pt_hbm, low_hbm, upp_hbm, sj_hbm, out_low_hbm, out_upp_hbm,
                 cols_v, sj_v, low_v, upp_v, olow_v, oupp_v, sem):
    c = lax.axis_index("c")
    s = lax.axis_index("s")
    wid = s * _NC + c
    q = wid % (_BLK // _BPW)
    blk = wid // (_BLK // _BPW)
    base = wid * _BPW
    iota = lax.iota(jnp.int32, _L)
    zero = jnp.zeros((_L,), jnp.float32)

    copy = pltpu.async_copy(pt_hbm.at[:, pl.ds(blk * _BLK, _BLK)], cols_v, sem)
    pltpu.sync_copy(sj_hbm.at[pl.ds(base, _BPW)], sj_v)
    pltpu.sync_copy(low_hbm.at[pl.ds(base, _BPW)], low_v)
    pltpu.sync_copy(upp_hbm.at[pl.ds(base, _BPW)], upp_v)
    copy.wait()

    for g in range(_NG):
        off = q * _BPW + g * _L
        sj = sj_v[pl.ds(g * _L, _L)]
        acc = zero
        for v in range(_VOCAB):
            p = cols_v[v, pl.ds(off, _L)]
            acc = acc + jnp.where(v < sj, p, zero)
        p_at = plsc.load_gather(cols_v, [sj, iota + off])
        low = low_v[pl.ds(g * _L, _L)]
        upp = upp_v[pl.ds(g * _L, _L)]
        rng = upp - low
        olow_v[pl.ds(g * _L, _L)] = low + rng * acc
        oupp_v[pl.ds(g * _L, _L)] = low + rng * (acc + p_at)

    pltpu.sync_copy(olow_v, out_low_hbm.at[pl.ds(base, _BPW)])
    pltpu.sync_copy(oupp_v, out_upp_hbm.at[pl.ds(base, _BPW)])


_sc_update_bounds = functools.partial(
    pl.kernel,
    mesh=plsc.VectorSubcoreMesh(core_axis_name="c", subcore_axis_name="s"),
    compiler_params=pltpu.CompilerParams(use_tc_tiling_on_sc=True,
                                         needs_layout_passes=False),
    out_type=(jax.ShapeDtypeStruct((_BATCH,), jnp.float32),
              jax.ShapeDtypeStruct((_BATCH,), jnp.float32)),
    scratch_types=[
        pltpu.VMEM((_VOCAB, _BLK), jnp.float32),
        pltpu.VMEM((_BPW,), jnp.int32),
        pltpu.VMEM((_BPW,), jnp.float32),
        pltpu.VMEM((_BPW,), jnp.float32),
        pltpu.VMEM((_BPW,), jnp.float32),
        pltpu.VMEM((_BPW,), jnp.float32),
        pltpu.SemaphoreType.DMA,
    ],
)(_bounds_body)


def kernel(low_bound, upp_bound, softmax, s_j):
    probs_t = _extract_t(softmax)
    sj = s_j.astype(jnp.int32)
    new_low, new_upp = _sc_update_bounds(probs_t, low_bound, upp_bound, sj)
    return (new_low, new_upp)


# speedup vs baseline: 1.0900x; 1.0362x over previous
import functools

import jax
import jax.numpy as jnp
from jax import lax
from jax.experimental import pallas as pl
from jax.experimental.pallas import tpu as pltpu
from jax.experimental.pallas import tpu_sc as plsc

_BATCH = 1024
_LAT_DIM = 64
_VOCAB = 256
_CUR_DIM = 32

_NC = 2
_NS = 16
_L = 16
_BBR = 64
_VHW = 128
_NG = _BBR // _L
_NK = _VHW // _L


def _bounds_body(sm_hbm, low_hbm, upp_hbm, sj_hbm, out_low_hbm, out_upp_hbm,
                 p3, sj_v, acc_v, res_v, low_v, upp_v, olow_v, oupp_v,
                 shared, sem):
    c = lax.axis_index("c")
    s = lax.axis_index("s")
    vh = s % 2
    bbw = s // 2 + c * (_NS // 2)
    base = bbw * _BBR
    iota = lax.iota(jnp.int32, _L)
    zero = jnp.zeros((_L,), jnp.float32)

    copy = pltpu.async_copy(
        sm_hbm.at[pl.ds(base, _BBR), pl.ds(_CUR_DIM, 1), pl.ds(vh * _VHW, _VHW)],
        p3, sem)
    pltpu.sync_copy(sj_hbm.at[pl.ds(base, _BBR)], sj_v)
    copy.wait()

    accL = [zero for _ in range(_NG)]
    accA = [zero for _ in range(_NG)]
    sjg = [sj_v[pl.ds(g * _L, _L)] for g in range(_NG)]
    for r in range(_BBR):
        vs = jnp.full((_L,), sjg[r // _L][r % _L], jnp.int32)
        alo = zero
        aat = zero
        for k in range(_NK):
            p = p3[r, 0, pl.ds(k * _L, _L)]
            vidx = iota + (vh * _VHW + k * _L)
            alo = alo + jnp.where(vidx < vs, p, zero)
            aat = aat + jnp.where(vidx == vs, p, zero)
        clo = jnp.full((_L,), jnp.sum(alo), jnp.float32)
        cat = jnp.full((_L,), jnp.sum(aat), jnp.float32)
        g, l = divmod(r, _L)
        lane = iota == l
        accL[g] = accL[g] + jnp.where(lane, clo, zero)
        accA[g] = accA[g] + jnp.where(lane, cat, zero)
    for g in range(_NG):
        acc_v[0, pl.ds(g * _L, _L)] = accL[g]
        acc_v[1, pl.ds(g * _L, _L)] = accA[g]

    pltpu.sync_copy(acc_v, shared.at[s])
    plsc.subcore_barrier()

    @pl.when(vh == 0)
    def _():
        pltpu.sync_copy(shared.at[s + 1], res_v)
        pltpu.sync_copy(low_hbm.at[pl.ds(base, _BBR)], low_v)
        pltpu.sync_copy(upp_hbm.at[pl.ds(base, _BBR)], upp_v)
        for g in range(_NG):
            cdf = accL[g] + res_v[0, pl.ds(g * _L, _L)]
            pat = accA[g] + res_v[1, pl.ds(g * _L, _L)]
            low = low_v[pl.ds(g * _L, _L)]
            upp = upp_v[pl.ds(g * _L, _L)]
            rng = upp - low
            olow_v[pl.ds(g * _L, _L)] = low + rng * cdf
            oupp_v[pl.ds(g * _L, _L)] = low + rng * (cdf + pat)
        pltpu.sync_copy(olow_v, out_low_hbm.at[pl.ds(base, _BBR)])
        pltpu.sync_copy(oupp_v, out_upp_hbm.at[pl.ds(base, _BBR)])


_sc_update_bounds = functools.partial(
    pl.kernel,
    mesh=plsc.VectorSubcoreMesh(core_axis_name="c", subcore_axis_name="s"),
    compiler_params=pltpu.CompilerParams(use_tc_tiling_on_sc=True,
                                         needs_layout_passes=False),
    out_type=(jax.ShapeDtypeStruct((_BATCH,), jnp.float32),
              jax.ShapeDtypeStruct((_BATCH,), jnp.float32)),
    scratch_types=[
        pltpu.VMEM((_BBR, 1, _VHW), jnp.float32),
        pltpu.VMEM((_BBR,), jnp.int32),
        pltpu.VMEM((2, _BBR), jnp.float32),
        pltpu.VMEM((2, _BBR), jnp.float32),
        pltpu.VMEM((_BBR,), jnp.float32),
        pltpu.VMEM((_BBR,), jnp.float32),
        pltpu.VMEM((_BBR,), jnp.float32),
        pltpu.VMEM((_BBR,), jnp.float32),
        pltpu.VMEM_SHARED((_NS, 2, _BBR), jnp.float32),
        pltpu.SemaphoreType.DMA,
    ],
)(_bounds_body)


def kernel(low_bound, upp_bound, softmax, s_j):
    sj = s_j.astype(jnp.int32)
    new_low, new_upp = _sc_update_bounds(softmax, low_bound, upp_bound, sj)
    return (new_low, new_upp)


# speedup vs baseline: 1.2076x; 1.1079x over previous
import functools

import jax
import jax.numpy as jnp
from jax import lax
from jax.experimental import pallas as pl
from jax.experimental.pallas import tpu as pltpu
from jax.experimental.pallas import tpu_sc as plsc

_BATCH = 1024
_LAT_DIM = 64
_VOCAB = 256
_CUR_DIM = 32

_NC = 2
_NS = 16
_L = 16
_NW = _NC * _NS
_BPW = _BATCH // _NW
_NG = _BPW // _L
_NK = _VOCAB // _L


def _bounds_body(sm_hbm, low_hbm, upp_hbm, sj_hbm, out_low_hbm, out_upp_hbm,
                 p3, sj_v, low_v, upp_v, olow_v, oupp_v, sem):
    c = lax.axis_index("c")
    s = lax.axis_index("s")
    wid = s * _NC + c
    base = wid * _BPW
    iota = lax.iota(jnp.int32, _L)
    zero = jnp.zeros((_L,), jnp.float32)
    zidx = jnp.zeros((_L,), jnp.int32)

    copy = pltpu.async_copy(
        sm_hbm.at[pl.ds(base, _BPW), pl.ds(_CUR_DIM, 1), pl.ds(0, _VOCAB)],
        p3, sem)
    pltpu.sync_copy(sj_hbm.at[pl.ds(base, _BPW)], sj_v)
    pltpu.sync_copy(low_hbm.at[pl.ds(base, _BPW)], low_v)
    pltpu.sync_copy(upp_hbm.at[pl.ds(base, _BPW)], upp_v)
    copy.wait()

    sjg = [sj_v[pl.ds(g * _L, _L)] for g in range(_NG)]
    accL = [zero for _ in range(_NG)]
    for r in range(_BPW):
        g, l = divmod(r, _L)
        vs = jnp.full((_L,), sjg[g][l], jnp.int32)
        alo = zero
        for k in range(_NK):
            p = p3[r, 0, pl.ds(k * _L, _L)]
            alo = alo + jnp.where(iota + (k * _L) < vs, p, zero)
        clo = jnp.full((_L,), jnp.sum(alo), jnp.float32)
        accL[g] = accL[g] + jnp.where(iota == l, clo, zero)

    for g in range(_NG):
        cdf = accL[g]
        pat = plsc.load_gather(p3, [iota + g * _L, zidx, sjg[g]])
        low = low_v[pl.ds(g * _L, _L)]
        upp = upp_v[pl.ds(g * _L, _L)]
        rng = upp - low
        olow_v[pl.ds(g * _L, _L)] = low + rng * cdf
        oupp_v[pl.ds(g * _L, _L)] = low + rng * (cdf + pat)

    pltpu.sync_copy(olow_v, out_low_hbm.at[pl.ds(base, _BPW)])
    pltpu.sync_copy(oupp_v, out_upp_hbm.at[pl.ds(base, _BPW)])


_sc_update_bounds = functools.partial(
    pl.kernel,
    mesh=plsc.VectorSubcoreMesh(core_axis_name="c", subcore_axis_name="s"),
    compiler_params=pltpu.CompilerParams(use_tc_tiling_on_sc=True,
                                         needs_layout_passes=False),
    out_type=(jax.ShapeDtypeStruct((_BATCH,), jnp.float32),
              jax.ShapeDtypeStruct((_BATCH,), jnp.float32)),
    scratch_types=[
        pltpu.VMEM((_BPW, 1, _VOCAB), jnp.float32),
        pltpu.VMEM((_BPW,), jnp.int32),
        pltpu.VMEM((_BPW,), jnp.float32),
        pltpu.VMEM((_BPW,), jnp.float32),
        pltpu.VMEM((_BPW,), jnp.float32),
        pltpu.VMEM((_BPW,), jnp.float32),
        pltpu.SemaphoreType.DMA,
    ],
)(_bounds_body)


def kernel(low_bound, upp_bound, softmax, s_j):
    sj = s_j.astype(jnp.int32)
    new_low, new_upp = _sc_update_bounds(softmax, low_bound, upp_bound, sj)
    return (new_low, new_upp)
